# baseline (speedup 1.0000x reference)
import functools

import jax
import jax.numpy as jnp
from jax import lax
from jax.experimental import pallas as pl
from jax.experimental.pallas import tpu as pltpu
from jax.experimental.pallas import tpu_sc as plsc

K = 65536
N_CLS = 1000
FEAT = 512
B = 1024
CPAD = 1024
NS = 16
ROWS_PER = K // NS
CHUNK = B // NS


def _control_body(lab_c, lab_r, inidx_c, tbl, ptr_c, kpc_c,
                  pos_out, src_out, vl_out, vi_out, ptr_out):
    labc = lab_c[...]
    labr = lab_r[...]
    ii = lax.broadcasted_iota(jnp.int32, (B, B), 0)
    jj = lax.broadcasted_iota(jnp.int32, (B, B), 1)
    eq = labc == labr
    intra = jnp.sum(jnp.where(eq & (ii > jj), 1, 0), axis=1, keepdims=True)

    ci = lax.broadcasted_iota(jnp.int32, (CPAD, B), 0)
    cnt = jnp.sum(jnp.where(ci == labr, 1, 0), axis=1, keepdims=True)
    ptr_out[...] = (ptr_c[...] + cnt) % kpc_c[...]

    cj = lax.broadcasted_iota(jnp.int32, (1, CPAD), 1)
    oh = jnp.where(labc == cj, 1.0, 0.0)
    g = jnp.round(jnp.dot(oh, tbl[...], preferred_element_type=jnp.float32,
                          precision=lax.Precision.HIGHEST))
    ptr_l = g[:, 0:1].astype(jnp.int32)
    start_l = g[:, 1:2].astype(jnp.int32)
    maxk_l = g[:, 2:3].astype(jnp.int32)

    offset = (ptr_l + intra) % jnp.maximum(maxk_l, 1)
    posv = start_l + offset
    maskv = intra < maxk_l
    bio = lax.broadcasted_iota(jnp.int32, (B, 1), 0)
    pos_out[...] = jnp.where(maskv, posv, posv[0:1, 0:1])
    src_out[...] = jnp.where(maskv, bio, 0)
    vl_out[...] = jnp.where(maskv, labc, labc[0:1, 0:1])
    vi_out[...] = jnp.where(maskv, inidx_c[...], inidx_c[0:1, 0:1])


def _control(labels, in_idx, queue_ptr, cls_start_idx, K_per_cls):
    lab_c = labels.reshape(B, 1)
    lab_r = labels.reshape(1, B)
    inidx_c = in_idx.astype(jnp.int32).reshape(B, 1)
    pad = CPAD - N_CLS
    tbl = jnp.pad(
        jnp.stack([queue_ptr, cls_start_idx, K_per_cls], axis=1).astype(jnp.float32),
        ((0, pad), (0, 128 - 3)))
    ptr_c = jnp.pad(queue_ptr, (0, pad)).reshape(CPAD, 1)
    kpc_c = jnp.pad(K_per_cls, (0, pad), constant_values=1).reshape(CPAD, 1)
    i32col = jax.ShapeDtypeStruct((B, 1), jnp.int32)
    outs = pl.pallas_call(
        _control_body,
        out_shape=[i32col, i32col, i32col, i32col,
                   jax.ShapeDtypeStruct((CPAD, 1), jnp.int32)],
    )(lab_c, lab_r, inidx_c, tbl, ptr_c, kpc_c)
    pos_eff, src_idx, val_l, val_i, new_ptr = outs
    return (pos_eff.reshape(B), src_idx.reshape(B), val_l.reshape(B),
            val_i.reshape(B), new_ptr.reshape(CPAD)[:N_CLS])


@functools.cache
def _make_sc_scatter():
    @functools.partial(
        pl.kernel,
        out_type=[
            jax.ShapeDtypeStruct((K, FEAT), jnp.float32),
            jax.ShapeDtypeStruct((K,), jnp.int32),
            jax.ShapeDtypeStruct((K,), jnp.int32),
        ],
        mesh=plsc.VectorSubcoreMesh(core_axis_name="c", subcore_axis_name="s",
                                    num_cores=1, num_subcores=NS),
        scratch_types=[
            pltpu.VMEM((CHUNK,), jnp.int32),
            pltpu.VMEM((CHUNK,), jnp.int32),
            pltpu.VMEM((CHUNK, FEAT), jnp.float32),
            pltpu.VMEM((CHUNK,), jnp.int32),
            pltpu.VMEM((CHUNK,), jnp.int32),
            pltpu.SemaphoreType.DMA,
            pltpu.SemaphoreType.DMA,
            pltpu.SemaphoreType.DMA,
            pltpu.SemaphoreType.DMA,
        ],
    )
    def _sc_scatter(keys_h, pos_h, src_h, vl_h, vi_h, qk_h, ql_h, qi_h,
                    ok_h, ol_h, oi_h,
                    posv, srcv, rowsv, vlv, viv, semk, seml, semi, semg):
        wid = lax.axis_index("s")
        r0 = wid * ROWS_PER
        ck = pltpu.async_copy(qk_h.at[pl.ds(r0, ROWS_PER)],
                              ok_h.at[pl.ds(r0, ROWS_PER)], semk)
        cl = pltpu.async_copy(ql_h.at[pl.ds(r0, ROWS_PER)],
                              ol_h.at[pl.ds(r0, ROWS_PER)], seml)
        ci = pltpu.async_copy(qi_h.at[pl.ds(r0, ROWS_PER)],
                              oi_h.at[pl.ds(r0, ROWS_PER)], semi)
        b0 = wid * CHUNK
        pltpu.sync_copy(pos_h.at[pl.ds(b0, CHUNK)], posv)
        pltpu.sync_copy(src_h.at[pl.ds(b0, CHUNK)], srcv)
        pltpu.sync_copy(vl_h.at[pl.ds(b0, CHUNK)], vlv)
        pltpu.sync_copy(vi_h.at[pl.ds(b0, CHUNK)], viv)
        pltpu.async_copy(keys_h.at[srcv], rowsv, semg).wait()
        ck.wait()
        cl.wait()
        ci.wait()
        plsc.subcore_barrier()
        pltpu.async_copy(rowsv, ok_h.at[posv], semg).wait()
        pltpu.async_copy(vlv, ol_h.at[posv], seml).wait()
        pltpu.async_copy(viv, oi_h.at[posv], semi).wait()

    return _sc_scatter


def kernel(keys, labels, in_idx, queue_k, queue_l, queue_i, queue_ptr,
           cls_start_idx, K_per_cls):
    pos_eff, src_idx, val_l, val_i, new_ptr = _control(
        labels, in_idx, queue_ptr, cls_start_idx, K_per_cls)
    ok, ol, oi = _make_sc_scatter()(keys, pos_eff, src_idx, val_l, val_i,
                                    queue_k, queue_l, queue_i)
    return ok, ol, oi.astype(queue_i.dtype), new_ptr


# speedup vs baseline: 23.1831x; 23.1831x over previous
import functools

import jax
import jax.numpy as jnp
from jax import lax
from jax.experimental import pallas as pl
from jax.experimental.pallas import tpu as pltpu
from jax.experimental.pallas import tpu_sc as plsc

K = 65536
N_CLS = 1000
FEAT = 512
B = 1024
CPAD = 1024
NC = 2
NS = 16
HALF = K // NC
ROWS_PER = K // (NC * NS)
CHUNK = B // NS
CR = 64
NCH = ROWS_PER // CR


def _control_body(lab_c, lab_r, inidx_c, tbl, ptr_c, kpc_c,
                  p0_out, s0_out, l0_out, i0_out,
                  p1_out, s1_out, l1_out, i1_out, ptr_out):
    labc = lab_c[...]
    labr = lab_r[...]
    ii = lax.broadcasted_iota(jnp.int32, (B, B), 0)
    jj = lax.broadcasted_iota(jnp.int32, (B, B), 1)
    tri = ii > jj
    eq = labc == labr
    intra = jnp.sum(jnp.where(eq & tri, 1, 0), axis=1, keepdims=True)

    ci = lax.broadcasted_iota(jnp.int32, (CPAD, B), 0)
    cnt = jnp.sum(jnp.where(ci == labr, 1, 0), axis=1, keepdims=True)
    ptr_out[...] = (ptr_c[...] + cnt) % kpc_c[...]

    cj = lax.broadcasted_iota(jnp.int32, (1, CPAD), 1)
    oh = jnp.where(labc == cj, 1.0, 0.0)
    g = jnp.round(jnp.dot(oh, tbl[...], preferred_element_type=jnp.float32,
                          precision=lax.Precision.HIGHEST))
    ptr_l = g[:, 0:1].astype(jnp.int32)
    start_l = g[:, 1:2].astype(jnp.int32)
    maxk_l = g[:, 2:3].astype(jnp.int32)

    offset = (ptr_l + intra) % jnp.maximum(maxk_l, 1)
    posv = start_l + offset
    maskv = intra < maxk_l
    bio = lax.broadcasted_iota(jnp.int32, (B, 1), 0)
    pos_eff = jnp.where(maskv, posv, posv[0:1, 0:1])
    src_eff = jnp.where(maskv, bio, 0)
    vl_eff = jnp.where(maskv, labc, labc[0:1, 0:1])
    vi_eff = jnp.where(maskv, inidx_c[...], inidx_c[0:1, 0:1])

    m0 = pos_eff < HALF
    m0_row = jnp.transpose(jnp.where(m0, 1, 0))
    rank0 = jnp.sum(jnp.where(tri & (m0_row > 0), 1, 0), axis=1, keepdims=True)
    rank_own = jnp.where(m0, rank0, bio - rank0)
    rank_row = jnp.transpose(rank_own)
    n0 = jnp.sum(jnp.where(m0, 1, 0))
    vals = jnp.concatenate(
        [pos_eff.astype(jnp.float32), src_eff.astype(jnp.float32),
         vl_eff.astype(jnp.float32), vi_eff.astype(jnp.float32),
         jnp.zeros((B, 124), jnp.float32)], axis=1)
    ent0 = (pos_eff[0:1, 0:1], src_eff[0:1, 0:1],
            vl_eff[0:1, 0:1], vi_eff[0:1, 0:1])

    def build(nh, member_row, outs):
        want = bio % jnp.maximum(nh, 1)
        perm = jnp.where((want == rank_row) & (member_row > 0), 1.0, 0.0)
        lst = jnp.round(jnp.dot(perm, vals,
                                preferred_element_type=jnp.float32,
                                precision=lax.Precision.HIGHEST))
        for col, (ref, pad) in enumerate(zip(outs, ent0)):
            v = lst[:, col:col + 1].astype(jnp.int32)
            ref[...] = jnp.where(nh > 0, v, pad)

    build(n0, m0_row, (p0_out, s0_out, l0_out, i0_out))
    build(B - n0, 1 - m0_row, (p1_out, s1_out, l1_out, i1_out))


def _control(labels, in_idx, queue_ptr, cls_start_idx, K_per_cls):
    lab_c = labels.reshape(B, 1)
    lab_r = labels.reshape(1, B)
    inidx_c = in_idx.astype(jnp.int32).reshape(B, 1)
    pad = CPAD - N_CLS
    tbl = jnp.pad(
        jnp.stack([queue_ptr, cls_start_idx, K_per_cls], axis=1).astype(jnp.float32),
        ((0, pad), (0, 128 - 3)))
    ptr_c = jnp.pad(queue_ptr, (0, pad)).reshape(CPAD, 1)
    kpc_c = jnp.pad(K_per_cls, (0, pad), constant_values=1).reshape(CPAD, 1)
    i32col = jax.ShapeDtypeStruct((B, 1), jnp.int32)
    outs = pl.pallas_call(
        _control_body,
        out_shape=[i32col] * 8 + [jax.ShapeDtypeStruct((CPAD, 1), jnp.int32)],
    )(lab_c, lab_r, inidx_c, tbl, ptr_c, kpc_c)
    lists = [o.reshape(B) for o in outs[:8]]
    new_ptr = outs[8].reshape(CPAD)[:N_CLS]
    pos_l = jnp.concatenate([lists[0], lists[4]])
    src_l = jnp.concatenate([lists[1], lists[5]])
    vl_l = jnp.concatenate([lists[2], lists[6]])
    vi_l = jnp.concatenate([lists[3], lists[7]])
    return pos_l, src_l, vl_l, vi_l, new_ptr


@functools.cache
def _make_sc_scatter():
    @functools.partial(
        pl.kernel,
        out_type=[
            jax.ShapeDtypeStruct((K, FEAT), jnp.float32),
            jax.ShapeDtypeStruct((K,), jnp.int32),
            jax.ShapeDtypeStruct((K,), jnp.int32),
        ],
        mesh=plsc.VectorSubcoreMesh(core_axis_name="c", subcore_axis_name="s",
                                    num_cores=NC, num_subcores=NS),
        scratch_types=[
            pltpu.VMEM((CR, FEAT), jnp.float32),
            pltpu.VMEM((CR, FEAT), jnp.float32),
            pltpu.VMEM((CHUNK,), jnp.int32),
            pltpu.VMEM((CHUNK,), jnp.int32),
            pltpu.VMEM((CHUNK, FEAT), jnp.float32),
            pltpu.VMEM((CHUNK,), jnp.int32),
            pltpu.VMEM((CHUNK,), jnp.int32),
            pltpu.SemaphoreType.DMA,
            pltpu.SemaphoreType.DMA,
            pltpu.SemaphoreType.DMA,
            pltpu.SemaphoreType.DMA,
            pltpu.SemaphoreType.DMA,
            pltpu.SemaphoreType.DMA,
            pltpu.SemaphoreType.DMA,
        ],
    )
    def _sc_scatter(keys_h, pos_h, src_h, vl_h, vi_h, qk_h, ql_h, qi_h,
                    ok_h, ol_h, oi_h,
                    bufa, bufb, posv, srcv, rowsv, vlv, viv,
                    semia, semib, semoa, semob, semg, seml, semi):
        c = lax.axis_index("c")
        s = lax.axis_index("s")
        gid = c * NS + s
        r0 = gid * ROWS_PER
        cl = pltpu.async_copy(ql_h.at[pl.ds(r0, ROWS_PER)],
                              ol_h.at[pl.ds(r0, ROWS_PER)], seml)
        ci = pltpu.async_copy(qi_h.at[pl.ds(r0, ROWS_PER)],
                              oi_h.at[pl.ds(r0, ROWS_PER)], semi)
        b0 = c * B + s * CHUNK
        pltpu.sync_copy(pos_h.at[pl.ds(b0, CHUNK)], posv)
        pltpu.sync_copy(src_h.at[pl.ds(b0, CHUNK)], srcv)
        pltpu.sync_copy(vl_h.at[pl.ds(b0, CHUNK)], vlv)
        pltpu.sync_copy(vi_h.at[pl.ds(b0, CHUNK)], viv)
        gat = pltpu.async_copy(keys_h.at[srcv], rowsv, semg)
        bufs = (bufa, bufb)
        sin = (semia, semib)
        sout = (semoa, semob)
        din = [None, None]
        dout = [None, None]
        din[0] = pltpu.async_copy(qk_h.at[pl.ds(r0, CR)], bufs[0], sin[0])
        for k in range(NCH):
            nk = k + 1
            if nk < NCH:
                if nk >= 2:
                    dout[nk % 2].wait()
                din[nk % 2] = pltpu.async_copy(
                    qk_h.at[pl.ds(r0 + nk * CR, CR)], bufs[nk % 2], sin[nk % 2])
            din[k % 2].wait()
            dout[k % 2] = pltpu.async_copy(
                bufs[k % 2], ok_h.at[pl.ds(r0 + k * CR, CR)], sout[k % 2])
        dout[(NCH - 2) % 2].wait()
        dout[(NCH - 1) % 2].wait()
        gat.wait()
        cl.wait()
        ci.wait()
        plsc.subcore_barrier()
        pltpu.async_copy(rowsv, ok_h.at[posv], semg).wait()
        pltpu.async_copy(vlv, ol_h.at[posv], seml).wait()
        pltpu.async_copy(viv, oi_h.at[posv], semi).wait()

    return _sc_scatter


def kernel(keys, labels, in_idx, queue_k, queue_l, queue_i, queue_ptr,
           cls_start_idx, K_per_cls):
    pos_l, src_l, vl_l, vi_l, new_ptr = _control(
        labels, in_idx, queue_ptr, cls_start_idx, K_per_cls)
    ok, ol, oi = _make_sc_scatter()(keys, pos_l, src_l, vl_l, vi_l,
                                    queue_k, queue_l, queue_i)
    return ok, ol, oi, new_ptr
